# baseline (speedup 1.0000x reference)
import functools

import jax
import jax.numpy as jnp
import numpy as np
from jax import lax
from jax.experimental import pallas as pl
from jax.experimental.pallas import tpu as pltpu
from jax.experimental.pallas import tpu_sc as plsc

VOCAB = 100000
D = 768
B = 4
S = 2048
N = B * S

NC, NS, L = 2, 16, 16
NW = NC * NS
S_W = S // NW
R = 32
NCHUNK = (B * S_W) // R
DV2 = D // (2 * L)
NBUF = 4


def _pe_table(max_len, d_model):
    pos = np.arange(max_len, dtype=np.float32)[:, None]
    i = np.arange(d_model, dtype=np.float32)[None, :]
    angle_rates = 1.0 / np.power(10000.0, (2.0 * np.floor(i / 2.0)) / float(d_model))
    angles = pos * angle_rates
    pe = np.zeros((max_len, d_model), dtype=np.float32)
    pe[:, 0::2] = np.sin(angles[:, 0::2])
    pe[:, 1::2] = np.cos(angles[:, 1::2])
    return pe


def _pe_packed(max_len, d_model):
    pe = _pe_table(max_len, d_model).reshape(max_len, d_model // 32, 2, 16)
    out = np.empty((max_len, d_model // 32, 32), dtype=np.float32)
    out[:, :, 0::2] = pe[:, :, 0, :]
    out[:, :, 1::2] = pe[:, :, 1, :]
    return out.reshape(max_len, d_model)


def _pe_packed_i32(max_len, d_model):
    import ml_dtypes
    pe_bf = _pe_packed(max_len, d_model).astype(ml_dtypes.bfloat16)
    return pe_bf.view(np.uint32).astype(np.int32).reshape(max_len, d_model // 2)


_PE_I32 = _pe_packed_i32(S, D)

_mesh = plsc.VectorSubcoreMesh(core_axis_name="c", subcore_axis_name="s")


@functools.partial(
    pl.kernel,
    out_type=jax.ShapeDtypeStruct((N, D), jnp.float32),
    mesh=_mesh,
    compiler_params=pltpu.CompilerParams(needs_layout_passes=False),
    scratch_types=[
        pltpu.VMEM((S_W, D // 2), jnp.int32),
        pltpu.VMEM((B, S_W), jnp.int32),
        [pltpu.VMEM((R, D), jnp.float32) for _ in range(NBUF)],
        pltpu.SemaphoreType.DMA,
        [pltpu.SemaphoreType.DMA for _ in range(NBUF)],
        [pltpu.SemaphoreType.DMA for _ in range(NBUF)],
    ],
)
def _embed_kernel(table_hbm, idx_hbm, pe_hbm, out_hbm,
                  pe_v, idx_v, tok_v, sem_pre, sem_in, sem_out):
    wid = lax.axis_index("s") * NC + lax.axis_index("c")
    s0 = wid * S_W

    idx_copies = [
        pltpu.async_copy(idx_hbm.at[b, pl.ds(s0, S_W)], idx_v.at[b], sem_pre)
        for b in range(B)
    ]
    pe_copy = pltpu.async_copy(pe_hbm.at[pl.ds(s0, S_W)], pe_v, sem_pre)
    for c in idx_copies:
        c.wait()

    def issue_gather(i):
        b, h = i // 2, i % 2
        return pltpu.async_copy(
            table_hbm.at[idx_v.at[b, pl.ds(h * R, R)]], tok_v[i % NBUF],
            sem_in[i % NBUF])

    gathers = [issue_gather(0), issue_gather(1)]
    scatters = [None] * NCHUNK
    pe_copy.wait()

    for i in range(NCHUNK):
        j = i % NBUF
        gathers[i].wait()
        pe_off = (i % 2) * R
        tok = tok_v[j]

        @plsc.parallel_loop(0, R)
        def add_row(r):
            prow = pe_off + r

            def add_group(k, carry):
                v = pe_v[prow, pl.ds(pl.multiple_of(k * L, L), L)]
                lo = plsc.bitcast(v << 16, jnp.float32)
                hi = plsc.bitcast(v & jnp.int32(-65536), jnp.float32)
                sl_lo = pl.ds(pl.multiple_of(k * 2 * L, 2 * L), L)
                sl_hi = pl.ds(pl.multiple_of(k * 2 * L + L, L), L)
                tok[r, sl_lo] = tok[r, sl_lo] + lo
                tok[r, sl_hi] = tok[r, sl_hi] + hi
                return carry

            lax.fori_loop(0, DV2, add_group, 0, unroll=4)

        b, h = i // 2, i % 2
        scatters[i] = pltpu.async_copy(
            tok_v[j], out_hbm.at[pl.ds(b * S + s0 + h * R, R)], sem_out[j])
        if i + 2 < NCHUNK:
            if i >= 2:
                scatters[i - 2].wait()
            gathers.append(issue_gather(i + 2))

    for i in range(NCHUNK - 4, NCHUNK):
        scatters[i].wait()


@jax.jit
def _embed(x, table):
    pe = jnp.asarray(_PE_I32)
    out = _embed_kernel(table, x, pe)
    return out.reshape(B, S, D)


def kernel(x, table):
    return _embed(x, table)


# speedup vs baseline: 1.2897x; 1.2897x over previous
import functools

import jax
import jax.numpy as jnp
import numpy as np
from jax import lax
from jax.experimental import pallas as pl
from jax.experimental.pallas import tpu as pltpu
from jax.experimental.pallas import tpu_sc as plsc

VOCAB = 100000
D = 768
B = 4
S = 2048
N = B * S

NC, NS, L = 2, 16, 16
NW = NC * NS
S_W = S // NW
R = 32
NCHUNK = (B * S_W) // R
DV2 = D // (2 * L)
NBUF = 4


def _pe_table(max_len, d_model):
    pos = np.arange(max_len, dtype=np.float32)[:, None]
    i = np.arange(d_model, dtype=np.float32)[None, :]
    angle_rates = 1.0 / np.power(10000.0, (2.0 * np.floor(i / 2.0)) / float(d_model))
    angles = pos * angle_rates
    pe = np.zeros((max_len, d_model), dtype=np.float32)
    pe[:, 0::2] = np.sin(angles[:, 0::2])
    pe[:, 1::2] = np.cos(angles[:, 1::2])
    return pe


def _pe_packed(max_len, d_model):
    pe = _pe_table(max_len, d_model).reshape(max_len, d_model // 32, 2, 16)
    out = np.empty((max_len, d_model // 32, 32), dtype=np.float32)
    out[:, :, 0::2] = pe[:, :, 0, :]
    out[:, :, 1::2] = pe[:, :, 1, :]
    return out.reshape(max_len, d_model)


def _pe_packed_i32(max_len, d_model):
    import ml_dtypes
    pe_bf = _pe_packed(max_len, d_model).astype(ml_dtypes.bfloat16)
    return pe_bf.view(np.uint32).astype(np.int32).reshape(max_len, d_model // 2)


_PE_I32 = _pe_packed_i32(S, D)

_mesh = plsc.VectorSubcoreMesh(core_axis_name="c", subcore_axis_name="s")


@functools.partial(
    pl.kernel,
    out_type=jax.ShapeDtypeStruct((N, D), jnp.float32),
    mesh=_mesh,
    compiler_params=pltpu.CompilerParams(needs_layout_passes=False),
    scratch_types=[
        pltpu.VMEM((S_W, D // 2), jnp.int32),
        pltpu.VMEM((B, S_W), jnp.int32),
        [pltpu.VMEM((R, D), jnp.float32) for _ in range(NBUF)],
        pltpu.SemaphoreType.DMA,
        [pltpu.SemaphoreType.DMA for _ in range(NBUF)],
        [pltpu.SemaphoreType.DMA for _ in range(NBUF)],
    ],
)
def _embed_kernel(table_hbm, idx_hbm, pe_hbm, out_hbm,
                  pe_v, idx_v, tok_v, sem_pre, sem_in, sem_out):
    wid = lax.axis_index("s") * NC + lax.axis_index("c")
    s0 = wid * S_W

    idx_copies = [
        pltpu.async_copy(idx_hbm.at[b, pl.ds(s0, S_W)], idx_v.at[b], sem_pre)
        for b in range(B)
    ]
    pe_copy = pltpu.async_copy(pe_hbm.at[pl.ds(s0, S_W)], pe_v, sem_pre)
    for c in idx_copies:
        c.wait()

    def issue_gather(i):
        b, h = i // 2, i % 2
        return pltpu.async_copy(
            table_hbm.at[idx_v.at[b, pl.ds(h * R, R)]], tok_v[i % NBUF],
            sem_in[i % NBUF])

    gathers = [issue_gather(0), issue_gather(1)]
    scatters = [None] * NCHUNK
    pe_copy.wait()

    for i in range(NCHUNK):
        j = i % NBUF
        gathers[i].wait()
        pe_off = (i % 2) * R
        tok = tok_v[j]

        @plsc.parallel_loop(0, R, unroll=2)
        def add_row(r):
            prow = pe_off + r
            for k in range(DV2):
                v = pe_v[prow, pl.ds(k * L, L)]
                lo = plsc.bitcast(v << 16, jnp.float32)
                hi = plsc.bitcast(v & jnp.int32(-65536), jnp.float32)
                sl_lo = pl.ds(k * 2 * L, L)
                sl_hi = pl.ds(k * 2 * L + L, L)
                tok[r, sl_lo] = tok[r, sl_lo] + lo
                tok[r, sl_hi] = tok[r, sl_hi] + hi

        b, h = i // 2, i % 2
        scatters[i] = pltpu.async_copy(
            tok_v[j], out_hbm.at[pl.ds(b * S + s0 + h * R, R)], sem_out[j])
        if i + 2 < NCHUNK:
            if i >= 2:
                scatters[i - 2].wait()
            gathers.append(issue_gather(i + 2))

    for i in range(NCHUNK - 4, NCHUNK):
        scatters[i].wait()


@jax.jit
def _embed(x, table):
    pe = jnp.asarray(_PE_I32)
    out = _embed_kernel(table, x, pe)
    return out.reshape(B, S, D)


def kernel(x, table):
    return _embed(x, table)


# speedup vs baseline: 1.3290x; 1.0305x over previous
import functools

import jax
import jax.numpy as jnp
import numpy as np
from jax import lax
from jax.experimental import pallas as pl
from jax.experimental.pallas import tpu as pltpu
from jax.experimental.pallas import tpu_sc as plsc

VOCAB = 100000
D = 768
B = 4
S = 2048
N = B * S

NC, NS, L = 2, 16, 16
NW = NC * NS
S_W = S // NW
R = 32
NCHUNK = (B * S_W) // R
DV2 = D // (2 * L)
NBUF = 4


def _pe_table(max_len, d_model):
    pos = np.arange(max_len, dtype=np.float32)[:, None]
    i = np.arange(d_model, dtype=np.float32)[None, :]
    angle_rates = 1.0 / np.power(10000.0, (2.0 * np.floor(i / 2.0)) / float(d_model))
    angles = pos * angle_rates
    pe = np.zeros((max_len, d_model), dtype=np.float32)
    pe[:, 0::2] = np.sin(angles[:, 0::2])
    pe[:, 1::2] = np.cos(angles[:, 1::2])
    return pe


def _pe_packed(max_len, d_model):
    pe = _pe_table(max_len, d_model).reshape(max_len, d_model // 32, 2, 16)
    out = np.empty((max_len, d_model // 32, 32), dtype=np.float32)
    out[:, :, 0::2] = pe[:, :, 0, :]
    out[:, :, 1::2] = pe[:, :, 1, :]
    return out.reshape(max_len, d_model)


def _pe_packed_i32(max_len, d_model):
    import ml_dtypes
    pe_bf = _pe_packed(max_len, d_model).astype(ml_dtypes.bfloat16)
    return pe_bf.view(np.uint32).astype(np.int32).reshape(max_len, d_model // 2)


_PE_I32 = _pe_packed_i32(S, D)

_mesh = plsc.VectorSubcoreMesh(core_axis_name="c", subcore_axis_name="s")


@functools.partial(
    pl.kernel,
    out_type=jax.ShapeDtypeStruct((N, D), jnp.float32),
    mesh=_mesh,
    compiler_params=pltpu.CompilerParams(needs_layout_passes=False),
    scratch_types=[
        pltpu.VMEM((S_W, D // 2), jnp.int32),
        pltpu.VMEM((B, S_W), jnp.int32),
        [pltpu.VMEM((R, D), jnp.float32) for _ in range(NBUF)],
        pltpu.SemaphoreType.DMA,
        [pltpu.SemaphoreType.DMA for _ in range(NBUF)],
        [pltpu.SemaphoreType.DMA for _ in range(NBUF)],
    ],
)
def _embed_kernel(table_hbm, idx_hbm, pe_hbm, out_hbm,
                  pe_v, idx_v, tok_v, sem_pre, sem_in, sem_out):
    wid = lax.axis_index("s") * NC + lax.axis_index("c")
    s0 = wid * S_W

    idx_copies = [
        pltpu.async_copy(idx_hbm.at[b, pl.ds(s0, S_W)], idx_v.at[b], sem_pre)
        for b in range(B)
    ]
    pe_copy = pltpu.async_copy(pe_hbm.at[pl.ds(s0, S_W)], pe_v, sem_pre)
    for c in idx_copies:
        c.wait()

    def issue_gather(i):
        b, h = i // 2, i % 2
        return pltpu.async_copy(
            table_hbm.at[idx_v.at[b, pl.ds(h * R, R)]], tok_v[i % NBUF],
            sem_in[i % NBUF])

    gathers = [issue_gather(0), issue_gather(1)]
    scatters = [None] * NCHUNK
    pe_copy.wait()

    for i in range(NCHUNK):
        j = i % NBUF
        gathers[i].wait()
        pe_off = (i % 2) * R
        tok = tok_v[j]

        @plsc.parallel_loop(0, R)
        def add_row(r):
            prow = pe_off + r
            for k in range(DV2):
                v = pe_v[prow, pl.ds(k * L, L)]
                lo = plsc.bitcast(v << 16, jnp.float32)
                hi = plsc.bitcast(v & jnp.int32(-65536), jnp.float32)
                sl_lo = pl.ds(k * 2 * L, L)
                sl_hi = pl.ds(k * 2 * L + L, L)
                tok[r, sl_lo] = tok[r, sl_lo] + lo
                tok[r, sl_hi] = tok[r, sl_hi] + hi

        b, h = i // 2, i % 2
        scatters[i] = pltpu.async_copy(
            tok_v[j], out_hbm.at[pl.ds(b * S + s0 + h * R, R)], sem_out[j])
        if i + 2 < NCHUNK:
            if i >= 2:
                scatters[i - 2].wait()
            gathers.append(issue_gather(i + 2))

    for i in range(NCHUNK - 4, NCHUNK):
        scatters[i].wait()


@jax.jit
def _embed(x, table):
    pe = jnp.asarray(_PE_I32)
    out = _embed_kernel(table, x, pe)
    return out.reshape(B, S, D)


def kernel(x, table):
    return _embed(x, table)


# speedup vs baseline: 1.3332x; 1.0031x over previous
import functools

import jax
import jax.numpy as jnp
import numpy as np
from jax import lax
from jax.experimental import pallas as pl
from jax.experimental.pallas import tpu as pltpu
from jax.experimental.pallas import tpu_sc as plsc

VOCAB = 100000
D = 768
B = 4
S = 2048
N = B * S

NC, NS, L = 2, 16, 16
NW = NC * NS
S_W = S // NW
R = 32
NCHUNK = (B * S_W) // R
DV2 = D // (2 * L)
NBUF = 4


def _pe_table(max_len, d_model):
    pos = np.arange(max_len, dtype=np.float32)[:, None]
    i = np.arange(d_model, dtype=np.float32)[None, :]
    angle_rates = 1.0 / np.power(10000.0, (2.0 * np.floor(i / 2.0)) / float(d_model))
    angles = pos * angle_rates
    pe = np.zeros((max_len, d_model), dtype=np.float32)
    pe[:, 0::2] = np.sin(angles[:, 0::2])
    pe[:, 1::2] = np.cos(angles[:, 1::2])
    return pe


def _pe_packed(max_len, d_model):
    pe = _pe_table(max_len, d_model).reshape(max_len, d_model // 32, 2, 16)
    out = np.empty((max_len, d_model // 32, 32), dtype=np.float32)
    out[:, :, 0::2] = pe[:, :, 0, :]
    out[:, :, 1::2] = pe[:, :, 1, :]
    return out.reshape(max_len, d_model)


def _pe_packed_i32(max_len, d_model):
    import ml_dtypes
    pe_bf = _pe_packed(max_len, d_model).astype(ml_dtypes.bfloat16)
    return pe_bf.view(np.uint32).astype(np.int32).reshape(max_len, d_model // 2)


_PE_I32 = _pe_packed_i32(S, D)

_mesh = plsc.VectorSubcoreMesh(core_axis_name="c", subcore_axis_name="s")


@functools.partial(
    pl.kernel,
    out_type=jax.ShapeDtypeStruct((N, D), jnp.float32),
    mesh=_mesh,
    compiler_params=pltpu.CompilerParams(needs_layout_passes=False),
    scratch_types=[
        pltpu.VMEM((S_W, D // 2), jnp.int32),
        pltpu.VMEM((B, S_W), jnp.int32),
        pltpu.VMEM((NBUF * R, D), jnp.float32),
        pltpu.SemaphoreType.DMA,
        [pltpu.SemaphoreType.DMA for _ in range(NBUF)],
        [pltpu.SemaphoreType.DMA for _ in range(NBUF)],
    ],
)
def _embed_kernel(table_hbm, idx_hbm, pe_hbm, out_hbm,
                  pe_v, idx_v, tok_v, sem_pre, sem_in, sem_out):
    wid = lax.axis_index("s") * NC + lax.axis_index("c")
    s0 = wid * S_W

    idx_copies = [
        pltpu.async_copy(idx_hbm.at[b, pl.ds(s0, S_W)], idx_v.at[b], sem_pre)
        for b in range(B)
    ]
    pe_copy = pltpu.async_copy(pe_hbm.at[pl.ds(s0, S_W)], pe_v, sem_pre)
    for c in idx_copies:
        c.wait()

    def issue_gather(i):
        b, h = i // 2, i % 2
        return pltpu.async_copy(
            table_hbm.at[idx_v.at[b, pl.ds(h * R, R)]],
            tok_v.at[pl.ds((i % NBUF) * R, R)], sem_in[i % NBUF])

    gathers = [issue_gather(0), issue_gather(1)]
    scatters = [None] * NCHUNK
    pe_copy.wait()

    for i in range(NCHUNK):
        j = i % NBUF
        gathers[i].wait()
        pe_off = (i % 2) * R
        row0 = j * R

        @plsc.parallel_loop(0, R)
        def add_row(r):
            prow = pe_off + r
            trow = row0 + r
            for k in range(DV2):
                v = pe_v[prow, pl.ds(k * L, L)]
                lo = plsc.bitcast(v << 16, jnp.float32)
                hi = plsc.bitcast(v & jnp.int32(-65536), jnp.float32)
                sl_lo = pl.ds(k * 2 * L, L)
                sl_hi = pl.ds(k * 2 * L + L, L)
                tok_v[trow, sl_lo] = tok_v[trow, sl_lo] + lo
                tok_v[trow, sl_hi] = tok_v[trow, sl_hi] + hi

        b, h = i // 2, i % 2
        scatters[i] = pltpu.async_copy(
            tok_v.at[pl.ds(row0, R)],
            out_hbm.at[pl.ds(b * S + s0 + h * R, R)], sem_out[j])
        if i + 2 < NCHUNK:
            if i >= 2:
                scatters[i - 2].wait()
            gathers.append(issue_gather(i + 2))

    for i in range(NCHUNK - 4, NCHUNK):
        scatters[i].wait()


@jax.jit
def _embed(x, table):
    pe = jnp.asarray(_PE_I32)
    out = _embed_kernel(table, x, pe)
    return out.reshape(B, S, D)


def kernel(x, table):
    return _embed(x, table)
